# baseline (speedup 1.0000x reference)
import functools

import jax
import jax.numpy as jnp
from jax import lax
from jax.experimental import pallas as pl
from jax.experimental.pallas import tpu as pltpu
from jax.experimental.pallas import tpu_sc as plsc

L = 16


def _vsqrt(x):
    i = plsc.bitcast(x, jnp.int32)
    i = (i >> 1) + jnp.int32(0x1FBD1DF5)
    y = plsc.bitcast(i, jnp.float32)
    for _ in range(3):
        y = 0.5 * (y + x / y)
    return y


def kernel(heads, relations, tails, entity_emb, relation_emb):
    B = heads.shape[0]
    D = entity_emb.shape[1]
    info = plsc.get_sparse_core_info()
    NC, NS = info.num_cores, info.num_subcores
    NW = NC * NS
    BPW = B // NW
    CH = 128
    NCHUNK = BPW // CH
    assert BPW % CH == 0 and CH % L == 0 and B % (8 * NW) == 0

    mesh = plsc.VectorSubcoreMesh(core_axis_name="c", subcore_axis_name="s")

    @functools.partial(
        pl.kernel,
        out_type=jax.ShapeDtypeStruct((B,), jnp.float32),
        mesh=mesh,
        compiler_params=pltpu.CompilerParams(needs_layout_passes=False),
        scratch_types=[
            pltpu.VMEM((BPW,), jnp.int32),
            pltpu.VMEM((BPW,), jnp.int32),
            pltpu.VMEM((BPW,), jnp.int32),
            pltpu.VMEM((2, CH, D), jnp.float32),
            pltpu.VMEM((2, CH, D), jnp.float32),
            pltpu.VMEM((2, CH, D), jnp.float32),
            pltpu.VMEM((BPW,), jnp.float32),
            pltpu.SemaphoreType.DMA,
            pltpu.SemaphoreType.DMA,
        ],
    )
    def run(heads_h, rels_h, tails_h, ent_h, rel_h, out_h,
            idx_h, idx_r, idx_t, hbuf, rbuf, tbuf, out_v, sem0, sem1):
        wid = lax.axis_index("s") * NC + lax.axis_index("c")
        base = wid * BPW
        pltpu.sync_copy(heads_h.at[pl.ds(base, BPW)], idx_h)
        pltpu.sync_copy(rels_h.at[pl.ds(base, BPW)], idx_r)
        pltpu.sync_copy(tails_h.at[pl.ds(base, BPW)], idx_t)

        sems = (sem0, sem1)

        def fire(c):
            b = c % 2
            sl = pl.ds(c * CH, CH)
            sem = sems[b]
            return (
                pltpu.async_copy(ent_h.at[idx_h.at[sl]], hbuf.at[b], sem),
                pltpu.async_copy(rel_h.at[idx_r.at[sl]], rbuf.at[b], sem),
                pltpu.async_copy(ent_h.at[idx_t.at[sl]], tbuf.at[b], sem),
            )

        cps = fire(0)
        for c in range(NCHUNK):
            nxt = fire(c + 1) if c + 1 < NCHUNK else None
            for cp in cps:
                cp.wait()
            b = c % 2
            hb, rb, tb = hbuf.at[b], rbuf.at[b], tbuf.at[b]
            for g in range(CH // L):
                rowv = lax.iota(jnp.int32, L) + jnp.int32(g * L)

                def dbody(_, carry):
                    acc, dv = carry
                    vh = plsc.load_gather(hb, [rowv, dv])
                    vr = plsc.load_gather(rb, [rowv, dv])
                    vt = plsc.load_gather(tb, [rowv, dv])
                    diff = (vh + vr) - vt
                    return acc + diff * diff, dv + 1

                acc, _ = lax.fori_loop(
                    0, D, dbody,
                    (jnp.zeros((L,), jnp.float32), jnp.zeros((L,), jnp.int32)),
                )
                out_v[pl.ds(c * CH + g * L, L)] = _vsqrt(acc)
            cps = nxt

        pltpu.sync_copy(out_v, out_h.at[pl.ds(base, BPW)])

    return run(heads.astype(jnp.int32), relations.astype(jnp.int32),
               tails.astype(jnp.int32), entity_emb, relation_emb)


# speedup vs baseline: 3.4503x; 3.4503x over previous
import functools

import jax
import jax.numpy as jnp
from jax import lax
from jax.experimental import pallas as pl
from jax.experimental.pallas import tpu as pltpu
from jax.experimental.pallas import tpu_sc as plsc

L = 16


def _vsqrt(x):
    i = plsc.bitcast(x, jnp.int32)
    i = jnp.int32(0x5F3759DF) - (i >> 1)
    y = plsc.bitcast(i, jnp.float32)
    hx = 0.5 * x
    for _ in range(3):
        y = y * (1.5 - hx * y * y)
    return x * y


def kernel(heads, relations, tails, entity_emb, relation_emb):
    B = heads.shape[0]
    D = entity_emb.shape[1]
    info = plsc.get_sparse_core_info()
    NC, NS = info.num_cores, info.num_subcores
    NW = NC * NS
    BPW = B // NW
    CH = 128
    NCHUNK = BPW // CH
    assert BPW % CH == 0 and CH % L == 0 and B % (8 * NW) == 0

    mesh = plsc.VectorSubcoreMesh(core_axis_name="c", subcore_axis_name="s")

    @functools.partial(
        pl.kernel,
        out_type=jax.ShapeDtypeStruct((B,), jnp.float32),
        mesh=mesh,
        compiler_params=pltpu.CompilerParams(needs_layout_passes=False),
        scratch_types=[
            pltpu.VMEM((BPW,), jnp.int32),
            pltpu.VMEM((BPW,), jnp.int32),
            pltpu.VMEM((BPW,), jnp.int32),
            pltpu.VMEM((2, CH, D), jnp.float32),
            pltpu.VMEM((2, CH, D), jnp.float32),
            pltpu.VMEM((2, CH, D), jnp.float32),
            pltpu.VMEM((BPW,), jnp.float32),
            pltpu.SemaphoreType.DMA,
            pltpu.SemaphoreType.DMA,
        ],
    )
    def run(heads_h, rels_h, tails_h, ent_h, rel_h, out_h,
            idx_h, idx_r, idx_t, hbuf, rbuf, tbuf, out_v, sem0, sem1):
        wid = lax.axis_index("s") * NC + lax.axis_index("c")
        base = wid * BPW
        pltpu.sync_copy(heads_h.at[pl.ds(base, BPW)], idx_h)
        pltpu.sync_copy(rels_h.at[pl.ds(base, BPW)], idx_r)
        pltpu.sync_copy(tails_h.at[pl.ds(base, BPW)], idx_t)

        sems = (sem0, sem1)

        def fire(c):
            b = c % 2
            sl = pl.ds(c * CH, CH)
            sem = sems[b]
            return (
                pltpu.async_copy(ent_h.at[idx_h.at[sl]], hbuf.at[b], sem),
                pltpu.async_copy(rel_h.at[idx_r.at[sl]], rbuf.at[b], sem),
                pltpu.async_copy(ent_h.at[idx_t.at[sl]], tbuf.at[b], sem),
            )

        cps = fire(0)
        for c in range(NCHUNK):
            nxt = fire(c + 1) if c + 1 < NCHUNK else None
            for cp in cps:
                cp.wait()
            b = c % 2
            hb, rb, tb = hbuf.at[b], rbuf.at[b], tbuf.at[b]
            iota = lax.iota(jnp.int32, L)

            def gbody(g, _):
                rowv = iota + g * L

                def kbody(_, carry):
                    acc, dbase = carry
                    rot = iota
                    for _s in range(L):
                        dv = dbase + rot
                        vh = plsc.load_gather(hb, [rowv, dv])
                        vr = plsc.load_gather(rb, [rowv, dv])
                        vt = plsc.load_gather(tb, [rowv, dv])
                        diff = (vh + vr) - vt
                        acc = acc + diff * diff
                        rot = (rot + 1) & (L - 1)
                    return acc, dbase + L

                acc, _ = lax.fori_loop(
                    0, D // L, kbody,
                    (jnp.zeros((L,), jnp.float32), jnp.zeros((L,), jnp.int32)),
                )
                out_v[pl.ds(c * CH + g * L, L)] = _vsqrt(acc)
                return 0

            lax.fori_loop(0, CH // L, gbody, 0)
            cps = nxt

        pltpu.sync_copy(out_v, out_h.at[pl.ds(base, BPW)])

    return run(heads.astype(jnp.int32), relations.astype(jnp.int32),
               tails.astype(jnp.int32), entity_emb, relation_emb)


# speedup vs baseline: 3.5127x; 1.0181x over previous
import functools

import jax
import jax.numpy as jnp
from jax import lax
from jax.experimental import pallas as pl
from jax.experimental.pallas import tpu as pltpu
from jax.experimental.pallas import tpu_sc as plsc

L = 16


def _vsqrt(x):
    i = plsc.bitcast(x, jnp.int32)
    i = jnp.int32(0x5F3759DF) - (i >> 1)
    y = plsc.bitcast(i, jnp.float32)
    hx = 0.5 * x
    for _ in range(3):
        y = y * (1.5 - hx * y * y)
    return x * y


def kernel(heads, relations, tails, entity_emb, relation_emb):
    B = heads.shape[0]
    D = entity_emb.shape[1]
    info = plsc.get_sparse_core_info()
    NC, NS = info.num_cores, info.num_subcores
    NW = NC * NS
    BPW = B // NW
    CH = 128
    NCHUNK = BPW // CH
    assert BPW % CH == 0 and CH % L == 0 and B % (8 * NW) == 0

    mesh = plsc.VectorSubcoreMesh(core_axis_name="c", subcore_axis_name="s")

    @functools.partial(
        pl.kernel,
        out_type=jax.ShapeDtypeStruct((B,), jnp.float32),
        mesh=mesh,
        compiler_params=pltpu.CompilerParams(needs_layout_passes=False),
        scratch_types=[
            pltpu.VMEM((BPW,), jnp.int32),
            pltpu.VMEM((BPW,), jnp.int32),
            pltpu.VMEM((BPW,), jnp.int32),
            pltpu.VMEM((2, CH, D), jnp.float32),
            pltpu.VMEM((2, CH, D), jnp.float32),
            pltpu.VMEM((2, CH, D), jnp.float32),
            pltpu.VMEM((BPW,), jnp.float32),
            pltpu.SemaphoreType.DMA,
            pltpu.SemaphoreType.DMA,
        ],
    )
    def run(heads_h, rels_h, tails_h, ent_h, rel_h, out_h,
            idx_h, idx_r, idx_t, hbuf, rbuf, tbuf, out_v, sem0, sem1):
        wid = lax.axis_index("s") * NC + lax.axis_index("c")
        base = wid * BPW
        pltpu.sync_copy(heads_h.at[pl.ds(base, BPW)], idx_h)
        pltpu.sync_copy(rels_h.at[pl.ds(base, BPW)], idx_r)
        pltpu.sync_copy(tails_h.at[pl.ds(base, BPW)], idx_t)

        sems = (sem0, sem1)
        iota = lax.iota(jnp.int32, L)

        def fire(cc, b):
            sl = pl.ds(cc * CH, CH)
            sem = sems[b]
            pltpu.async_copy(ent_h.at[idx_h.at[sl]], hbuf.at[b], sem)
            pltpu.async_copy(rel_h.at[idx_r.at[sl]], rbuf.at[b], sem)
            pltpu.async_copy(ent_h.at[idx_t.at[sl]], tbuf.at[b], sem)

        def drain(b):
            pltpu.make_async_copy(ent_h.at[pl.ds(0, CH)], hbuf.at[b], sems[b]).wait()
            pltpu.make_async_copy(rel_h.at[pl.ds(0, CH)], rbuf.at[b], sems[b]).wait()
            pltpu.make_async_copy(ent_h.at[pl.ds(0, CH)], tbuf.at[b], sems[b]).wait()

        def compute(cc, b):
            hb, rb, tb = hbuf.at[b], rbuf.at[b], tbuf.at[b]

            def gbody(g, _):
                rowv = iota + g * L

                def kbody(_, carry):
                    acc, dbase = carry
                    rot = iota
                    for _s in range(L):
                        dv = dbase + rot
                        vh = plsc.load_gather(hb, [rowv, dv])
                        vr = plsc.load_gather(rb, [rowv, dv])
                        vt = plsc.load_gather(tb, [rowv, dv])
                        diff = (vh + vr) - vt
                        acc = acc + diff * diff
                        rot = (rot + 1) & (L - 1)
                    return acc, dbase + L

                acc, _ = lax.fori_loop(
                    0, D // L, kbody,
                    (jnp.zeros((L,), jnp.float32), jnp.zeros((L,), jnp.int32)),
                )
                out_v[pl.ds(cc * CH + g * L, L)] = _vsqrt(acc)
                return 0

            lax.fori_loop(0, CH // L, gbody, 0)

        fire(jnp.int32(0), 0)

        def pair(i, _):
            cc = i * 2
            fire(cc + 1, 1)
            drain(0)
            compute(cc, 0)

            @pl.when(cc + 2 < NCHUNK)
            def _():
                fire(cc + 2, 0)

            drain(1)
            compute(cc + 1, 1)
            return 0

        lax.fori_loop(0, NCHUNK // 2, pair, 0)

        pltpu.sync_copy(out_v, out_h.at[pl.ds(base, BPW)])

    return run(heads.astype(jnp.int32), relations.astype(jnp.int32),
               tails.astype(jnp.int32), entity_emb, relation_emb)
